# baseline (speedup 1.0000x reference)
import jax
import jax.numpy as jnp
from jax.experimental import pallas as pl
from jax.experimental.pallas import tpu as pltpu

_B, _N, _D, _R = 16384, 20, 64, 8
_BS = 512
_TB = _BS * _N
_GB = 16
_NG = _BS // _GB
_K = 4
_CB = _B // _K


def _body(u_ref, i_ref, ao_ref, s_ref, waor_ref, wuir_ref, rp_ref,
          out_ref, c_ref, v_ref):
    @pl.when(pl.program_id(0) == 0)
    def _():
        for r in range(_R):
            wu = wuir_ref[r]
            wa = waor_ref[r]
            c_ref[:, r * 128:(r + 1) * 128] = jax.lax.dot_general(
                wu, wa, (((1,), (1,)), ((), ())),
                preferred_element_type=jnp.float32)
            v_ref[r:r + 1, :] = jnp.sum(
                wa * rp_ref[r:r + 1, :], axis=-1).reshape(1, 2 * _D)

    u = u_ref[:]
    i = i_ref[:]
    g = (jnp.dot(u, c_ref[:_D, :], preferred_element_type=jnp.float32)
         + jnp.dot(i, c_ref[_D:, :], preferred_element_type=jnp.float32))

    parts = []
    for r in range(_R):
        parts.append(
            (g[:, r * 128:(r + 1) * 128] + v_ref[r:r + 1, :])
            .astype(jnp.bfloat16)[None])
    gstack = jnp.concatenate(parts, axis=0)

    s3 = jax.lax.broadcast_in_dim(s_ref[:], (_BS, _N, _R * _GB), (0, 1))
    bmod = jax.lax.broadcasted_iota(jnp.int32, (_BS, _N, _R * _GB), 0) % _GB
    lane = jax.lax.broadcasted_iota(jnp.int32, (_BS, _N, _R * _GB), 2)
    oh3 = (lane == s3 * _GB + bmod).astype(jnp.float32)
    ohbd = oh3.reshape(_TB, _R * _GB).astype(jnp.bfloat16)

    gsels = []
    for gi in range(_NG):
        rhs = gstack[:, gi * _GB:(gi + 1) * _GB, :].reshape(_R * _GB, 2 * _D)
        lhs = ohbd[gi * _GB * _N:(gi + 1) * _GB * _N, :]
        gsels.append(jax.lax.dot_general(
            lhs, rhs, (((1,), (0,)), ((), ())),
            preferred_element_type=jnp.float32))
    gsel = jnp.concatenate(gsels, axis=0)

    prod = ao_ref[:] * gsel

    ones = jnp.ones((1, 2 * _D), jnp.float32)
    preds = []
    for c in range(_TB // 128):
        preds.append(jax.lax.dot_general(
            ones, prod[c * 128:(c + 1) * 128, :],
            (((1,), (1,)), ((), ())), preferred_element_type=jnp.float32))
    out_ref[...] = jnp.concatenate(preds, axis=0)


def _chunk_call(u_k, i_k, ao2_k, s_k, w_aor, w_uir, r_param):
    grid = (_CB // _BS,)
    return pl.pallas_call(
        _body,
        grid=grid,
        in_specs=[
            pl.BlockSpec((_BS, _D), lambda i: (i, 0)),
            pl.BlockSpec((_BS, _D), lambda i: (i, 0)),
            pl.BlockSpec((_TB, 2 * _D), lambda i: (i, 0)),
            pl.BlockSpec((_BS, _N), lambda i: (i, 0)),
            pl.BlockSpec((_R, 2 * _D, _D), lambda i: (0, 0, 0)),
            pl.BlockSpec((_R, 2 * _D, _D), lambda i: (0, 0, 0)),
            pl.BlockSpec((_R, _D), lambda i: (0, 0)),
        ],
        out_specs=pl.BlockSpec((_TB // 128, 128), lambda i: (i, 0)),
        out_shape=jax.ShapeDtypeStruct((_CB * _N // 128, 128), jnp.float32),
        scratch_shapes=[
            pltpu.VMEM((2 * _D, _R * 2 * _D), jnp.float32),
            pltpu.VMEM((_R, 2 * _D), jnp.float32),
        ],
    )(u_k, i_k, ao2_k, s_k, w_aor, w_uir, r_param)


def kernel(u_emb, i_emb, a_emb, o_emb, s, w_aor, w_uir, r_param):
    outs = []
    for k in range(_K):
        lo = k * _CB
        ao2_k = jnp.concatenate(
            [a_emb[lo:lo + _CB], o_emb[lo:lo + _CB]], axis=-1
        ).reshape(_CB * _N, 2 * _D)
        outs.append(_chunk_call(
            u_emb[lo:lo + _CB], i_emb[lo:lo + _CB], ao2_k, s[lo:lo + _CB],
            w_aor, w_uir, r_param))
    return jnp.concatenate(outs, axis=0).reshape(_B, _N)


# speedup vs baseline: 5.4358x; 5.4358x over previous
import jax
import jax.numpy as jnp
from jax.experimental import pallas as pl
from jax.experimental.pallas import tpu as pltpu

_B, _N, _D, _R = 16384, 20, 64, 8
_BS = 512
_KA = 2 * _D + 8


def _body(u_ref, i_ref, a_ref, o_ref, s_ref, waor_ref, wuir_ref, rp_ref,
          out_ref, ct_ref):
    @pl.when(pl.program_id(0) == 0)
    def _():
        for r in range(_R):
            wa = waor_ref[r]
            wu = wuir_ref[r]
            ct_ref[r * 128:(r + 1) * 128, :2 * _D] = jax.lax.dot_general(
                wa, wu, (((1,), (1,)), ((), ())),
                preferred_element_type=jnp.float32)
            vcol = jax.lax.dot_general(
                wa, rp_ref[r:r + 1, :], (((1,), (1,)), ((), ())),
                preferred_element_type=jnp.float32)
            ct_ref[r * 128:(r + 1) * 128, 2 * _D:] = jnp.concatenate(
                [vcol, jnp.zeros((2 * _D, _KA - 2 * _D - 1), jnp.float32)],
                axis=1)

    ui_aug = jnp.concatenate(
        [u_ref[:], i_ref[:], jnp.ones((_KA - 2 * _D, _BS), jnp.float32)],
        axis=0)
    g = jax.lax.dot_general(
        ct_ref[:], ui_aug, (((1,), (0,)), ((), ())),
        preferred_element_type=jnp.float32)

    s_all = s_ref[:]
    for n in range(_N):
        sn = s_all[n:n + 1, :]
        acc = g[0:2 * _D, :]
        for r in range(1, _R):
            acc = jnp.where(sn == r, g[r * 128:(r + 1) * 128, :], acc)
        t = a_ref[n] * acc[:_D, :] + o_ref[n] * acc[_D:, :]
        out_ref[n:n + 1, :] = jnp.sum(t, axis=0, keepdims=True)


def kernel(u_emb, i_emb, a_emb, o_emb, s, w_aor, w_uir, r_param):
    uT = u_emb.T
    iT = i_emb.T
    aT = jnp.transpose(a_emb, (1, 2, 0))
    oT = jnp.transpose(o_emb, (1, 2, 0))
    sT = s.T

    grid = (_B // _BS,)
    outT = pl.pallas_call(
        _body,
        grid=grid,
        in_specs=[
            pl.BlockSpec((_D, _BS), lambda i: (0, i)),
            pl.BlockSpec((_D, _BS), lambda i: (0, i)),
            pl.BlockSpec((_N, _D, _BS), lambda i: (0, 0, i)),
            pl.BlockSpec((_N, _D, _BS), lambda i: (0, 0, i)),
            pl.BlockSpec((_N, _BS), lambda i: (0, i)),
            pl.BlockSpec((_R, 2 * _D, _D), lambda i: (0, 0, 0)),
            pl.BlockSpec((_R, 2 * _D, _D), lambda i: (0, 0, 0)),
            pl.BlockSpec((_R, _D), lambda i: (0, 0)),
        ],
        out_specs=pl.BlockSpec((_N, _BS), lambda i: (0, i)),
        out_shape=jax.ShapeDtypeStruct((_N, _B), jnp.float32),
        scratch_shapes=[
            pltpu.VMEM((_R * 2 * _D, _KA), jnp.float32),
        ],
    )(uT, iT, aT, oT, sT, w_aor, w_uir, r_param)
    return outT.T


# speedup vs baseline: 5.4397x; 1.0007x over previous
import jax
import jax.numpy as jnp
from jax.experimental import pallas as pl
from jax.experimental.pallas import tpu as pltpu

_B, _N, _D, _R = 16384, 20, 64, 8
_BS = 1024
_KA = 2 * _D + 8


def _body(u_ref, i_ref, a_ref, o_ref, s_ref, waor_ref, wuir_ref, rp_ref,
          out_ref, ct_ref):
    @pl.when(pl.program_id(0) == 0)
    def _():
        for r in range(_R):
            wa = waor_ref[r]
            wu = wuir_ref[r]
            ct_ref[r * 128:(r + 1) * 128, :2 * _D] = jax.lax.dot_general(
                wa, wu, (((1,), (1,)), ((), ())),
                preferred_element_type=jnp.float32)
            vcol = jax.lax.dot_general(
                wa, rp_ref[r:r + 1, :], (((1,), (1,)), ((), ())),
                preferred_element_type=jnp.float32)
            ct_ref[r * 128:(r + 1) * 128, 2 * _D:] = jnp.concatenate(
                [vcol, jnp.zeros((2 * _D, _KA - 2 * _D - 1), jnp.float32)],
                axis=1)

    ui_aug = jnp.concatenate(
        [u_ref[:], i_ref[:], jnp.ones((_KA - 2 * _D, _BS), jnp.float32)],
        axis=0)
    g = jax.lax.dot_general(
        ct_ref[:], ui_aug, (((1,), (0,)), ((), ())),
        preferred_element_type=jnp.float32)

    s_all = s_ref[:]
    for n in range(_N):
        sn = s_all[n:n + 1, :]
        acc = g[0:2 * _D, :]
        for r in range(1, _R):
            acc = jnp.where(sn == r, g[r * 128:(r + 1) * 128, :], acc)
        t = a_ref[n] * acc[:_D, :] + o_ref[n] * acc[_D:, :]
        out_ref[n:n + 1, :] = jnp.sum(t, axis=0, keepdims=True)


def kernel(u_emb, i_emb, a_emb, o_emb, s, w_aor, w_uir, r_param):
    uT = u_emb.T
    iT = i_emb.T
    aT = jnp.transpose(a_emb, (1, 2, 0))
    oT = jnp.transpose(o_emb, (1, 2, 0))
    sT = s.T

    grid = (_B // _BS,)
    outT = pl.pallas_call(
        _body,
        grid=grid,
        in_specs=[
            pl.BlockSpec((_D, _BS), lambda i: (0, i)),
            pl.BlockSpec((_D, _BS), lambda i: (0, i)),
            pl.BlockSpec((_N, _D, _BS), lambda i: (0, 0, i)),
            pl.BlockSpec((_N, _D, _BS), lambda i: (0, 0, i)),
            pl.BlockSpec((_N, _BS), lambda i: (0, i)),
            pl.BlockSpec((_R, 2 * _D, _D), lambda i: (0, 0, 0)),
            pl.BlockSpec((_R, 2 * _D, _D), lambda i: (0, 0, 0)),
            pl.BlockSpec((_R, _D), lambda i: (0, 0)),
        ],
        out_specs=pl.BlockSpec((_N, _BS), lambda i: (0, i)),
        out_shape=jax.ShapeDtypeStruct((_N, _B), jnp.float32),
        scratch_shapes=[
            pltpu.VMEM((_R * 2 * _D, _KA), jnp.float32),
        ],
    )(uT, iT, aT, oT, sT, w_aor, w_uir, r_param)
    return outT.T
